# Initial kernel scaffold; baseline (speedup 1.0000x reference)
import jax
import jax.numpy as jnp
from jax.experimental import pallas as pl


def kernel(seeds, nbr1, nbr2, emb, Wq1, Wk1, Wv1, Ws1, Wq2, Wk2, Wv2, Ws2):
    raise NotImplementedError("write your pallas kernel here")


# speedup vs baseline: 4.5160x; 4.5160x over previous
import functools

import jax
import jax.numpy as jnp
from jax import lax
from jax.experimental import pallas as pl
from jax.experimental.pallas import tpu as pltpu
from jax.experimental.pallas import tpu_sc as plsc

B = 2048
FAN1 = 16
FAN2 = 16
EMB = 128
HID = 128
HEADS = 8
HD = HID // HEADS

NW = 32
CHUNK = 128


def _sc_gather_all(emb, idx2, idx1, idx0):
    n2 = idx2.shape[0] // NW
    n1 = idx1.shape[0] // NW
    mesh = plsc.VectorSubcoreMesh(core_axis_name="c", subcore_axis_name="s")

    @functools.partial(
        pl.kernel,
        mesh=mesh,
        out_type=[
            jax.ShapeDtypeStruct((idx2.size, EMB), jnp.float32),
            jax.ShapeDtypeStruct((idx1.size, EMB), jnp.float32),
            jax.ShapeDtypeStruct((idx0.size, EMB), jnp.float32),
        ],
        scratch_types=[
            pltpu.VMEM((n2, CHUNK), jnp.int32),
            pltpu.VMEM((n1, CHUNK), jnp.int32),
            pltpu.VMEM((64,), jnp.int32),
            pltpu.VMEM((CHUNK, EMB), jnp.float32),
            pltpu.VMEM((CHUNK, EMB), jnp.float32),
            pltpu.VMEM((64, EMB), jnp.float32),
            pltpu.SemaphoreType.DMA,
            pltpu.SemaphoreType.DMA,
            pltpu.SemaphoreType.DMA,
        ],
    )
    def k(emb_hbm, idx2_hbm, idx1_hbm, idx0_hbm, out2_hbm, out1_hbm, out0_hbm,
          idx2_v, idx1_v, idx0_v, rows_a, rows_b, rows_s, sem_a, sem_b, sem_s):
        wid = lax.axis_index("s") * 2 + lax.axis_index("c")

        pltpu.sync_copy(idx2_hbm.at[pl.ds(wid * n2, n2)], idx2_v)
        pltpu.sync_copy(idx1_hbm.at[pl.ds(wid * n1, n1)], idx1_v)
        pltpu.sync_copy(idx0_hbm.at[wid], idx0_v)

        base2 = wid * n2 * CHUNK

        def body(j, carry):
            del carry
            j2 = j * 2
            ca = pltpu.async_copy(emb_hbm.at[idx2_v.at[j2]], rows_a, sem_a)
            cb = pltpu.async_copy(emb_hbm.at[idx2_v.at[j2 + 1]], rows_b, sem_b)
            ca.wait()
            pltpu.sync_copy(rows_a, out2_hbm.at[pl.ds(base2 + j2 * CHUNK, CHUNK)])
            cb.wait()
            pltpu.sync_copy(rows_b, out2_hbm.at[pl.ds(base2 + (j2 + 1) * CHUNK, CHUNK)])
            return 0

        lax.fori_loop(0, n2 // 2, body, 0)

        base1 = wid * n1 * CHUNK

        def body1(j, carry):
            del carry
            pltpu.async_copy(emb_hbm.at[idx1_v.at[j]], rows_a, sem_a).wait()
            pltpu.sync_copy(rows_a, out1_hbm.at[pl.ds(base1 + j * CHUNK, CHUNK)])
            return 0

        lax.fori_loop(0, n1, body1, 0)

        pltpu.async_copy(emb_hbm.at[idx0_v], rows_s, sem_s).wait()
        pltpu.sync_copy(rows_s, out0_hbm.at[pl.ds(wid * 64, 64)])

    return k(emb, idx2, idx1, idx0)


def _head_matrix():
    d = lax.broadcasted_iota(jnp.int32, (HID, HEADS), 0)
    h = lax.broadcasted_iota(jnp.int32, (HID, HEADS), 1)
    return (d // HD == h).astype(jnp.float32)


def _gat_block(hs, hn, wq, wk, wv, ws, n, f):
    scale = float(HD) ** (-0.5)
    dn = (((1,), (1,)), ((), ()))
    q = lax.dot_general(hs, wq, dn, preferred_element_type=jnp.float32)
    k = lax.dot_general(hn, wk, dn, preferred_element_type=jnp.float32)
    v = lax.dot_general(hn, wv, dn, preferred_element_type=jnp.float32)
    qr = jnp.broadcast_to(q[:, None, :], (n, f, HID)).reshape(n * f, HID)
    p = k * qr * scale
    S = _head_matrix()
    scores = lax.dot_general(p, S, (((1,), (0,)), ((), ())),
                             preferred_element_type=jnp.float32)
    s3 = scores.reshape(n, f, HEADS)
    m = jnp.max(s3, axis=1)
    e = jnp.exp(s3 - m[:, None, :])
    den = jnp.sum(e, axis=1)
    attn = (e / den[:, None, :]).reshape(n * f, HEADS)
    w2d = lax.dot_general(attn, S, (((1,), (1,)), ((), ())),
                          preferred_element_type=jnp.float32)
    agg = (w2d * v).reshape(n, f, HID).sum(axis=1)
    return lax.dot_general(hs, ws, dn, preferred_element_type=jnp.float32) + agg


SEED_BLK = 16


def _fused_gat_kernel(h0_ref, h1s_ref, h2_ref, wq1_ref, wk1_ref, wv1_ref,
                      ws1_ref, wq2_ref, wk2_ref, wv2_ref, ws2_ref, out_ref):
    n1 = SEED_BLK * FAN1
    h1s = h1s_ref[...]
    h2 = h2_ref[...]
    h1 = _gat_block(h1s, h2, wq1_ref[...], wk1_ref[...], wv1_ref[...],
                    ws1_ref[...], n1, FAN2)
    h1 = jnp.maximum(h1, 0.0)
    h0 = h0_ref[...]
    out = _gat_block(h0, h1, wq2_ref[...], wk2_ref[...], wv2_ref[...],
                     ws2_ref[...], SEED_BLK, FAN1)
    out_ref[...] = jnp.maximum(out, 0.0)


def _tc_fused(h0, h1s, h2, Wq1, Wk1, Wv1, Ws1, Wq2, Wk2, Wv2, Ws2):
    grid = B // SEED_BLK
    wspec = pl.BlockSpec((HID, HID), lambda i: (0, 0))
    return pl.pallas_call(
        _fused_gat_kernel,
        grid=(grid,),
        in_specs=[
            pl.BlockSpec((SEED_BLK, EMB), lambda i: (i, 0)),
            pl.BlockSpec((SEED_BLK * FAN1, EMB), lambda i: (i, 0)),
            pl.BlockSpec((SEED_BLK * FAN1 * FAN2, EMB), lambda i: (i, 0)),
            wspec, wspec, wspec, wspec, wspec, wspec, wspec, wspec,
        ],
        out_specs=pl.BlockSpec((SEED_BLK, HID), lambda i: (i, 0)),
        out_shape=jax.ShapeDtypeStruct((B, HID), jnp.float32),
    )(h0, h1s, h2, Wq1, Wk1, Wv1, Ws1, Wq2, Wk2, Wv2, Ws2)


def kernel(seeds, nbr1, nbr2, emb, Wq1, Wk1, Wv1, Ws1, Wq2, Wk2, Wv2, Ws2):
    idx2 = nbr2.reshape(B * FAN1 * FAN2 // CHUNK, CHUNK).astype(jnp.int32)
    idx1 = nbr1.reshape(B * FAN1 // CHUNK, CHUNK).astype(jnp.int32)
    idx0 = seeds.reshape(NW, B // NW).astype(jnp.int32)
    h2, h1s, h0 = _sc_gather_all(emb, idx2, idx1, idx0)
    return _tc_fused(h0, h1s, h2, Wq1, Wk1, Wv1, Ws1, Wq2, Wk2, Wv2, Ws2)
